# Initial kernel scaffold; baseline (speedup 1.0000x reference)
import jax
import jax.numpy as jnp
from jax.experimental import pallas as pl


def kernel(x, edge_index, W1, b1, gamma, beta, W2, b2):
    raise NotImplementedError("write your pallas kernel here")


# speedup vs baseline: 6.6795x; 6.6795x over previous
import functools

import jax
import jax.numpy as jnp
from jax import lax
from jax.experimental import pallas as pl
from jax.experimental.pallas import tpu as pltpu
from jax.experimental.pallas import tpu_sc as plsc

_N = 10000
_E = 320000
_D = 128

_NC = 2
_NS = 16
_NW = _NC * _NS
_CH = 128
_CHUNKS = _E // _CH
_BASE_CHUNKS = _CHUNKS // _NW
_EXTRA = _CHUNKS - _BASE_CHUNKS * _NW
_ROWS_MAIN = 624
_ROWS_LAST = _N - 15 * _ROWS_MAIN
_ZROWS = 208


def _agg_body(x_hbm, src_hbm, dst_hbm, out_hbm, acc_sh, src_v, dst_v, rows_v,
              zero_v, gsem):
    c = lax.axis_index("c")
    s = lax.axis_index("s")
    wid = c * _NS + s

    zvec = jnp.zeros((16,), jnp.float32)

    def zrow(i, carry):
        for j in range(8):
            zero_v[i, pl.ds(j * 16, 16)] = zvec
        return carry

    lax.fori_loop(0, _ZROWS, zrow, 0)
    rbase = s * _ROWS_MAIN
    for k in range(_ROWS_MAIN // _ZROWS):
        pltpu.sync_copy(zero_v, acc_sh.at[pl.ds(rbase + k * _ZROWS, _ZROWS)])

    @pl.when(s == _NS - 1)
    def _zero_tail():
        pltpu.sync_copy(zero_v.at[pl.ds(0, _ROWS_LAST - _ROWS_MAIN)],
                        acc_sh.at[pl.ds(rbase + _ROWS_MAIN,
                                        _ROWS_LAST - _ROWS_MAIN)])

    plsc.subcore_barrier()

    nchunks = _BASE_CHUNKS + jnp.where(wid < _EXTRA, 1, 0)

    def chunk(j, carry):
        ebase = (wid + j * _NW) * _CH
        pltpu.sync_copy(src_hbm.at[pl.ds(ebase, _CH)], src_v)
        pltpu.sync_copy(dst_hbm.at[pl.ds(ebase, _CH)], dst_v)
        pltpu.async_copy(x_hbm.at[src_v], rows_v, gsem).wait()
        pltpu.sync_copy(rows_v, acc_sh.at[dst_v], add=True)
        return carry

    lax.fori_loop(0, nchunks, chunk, 0)
    plsc.subcore_barrier()

    pltpu.sync_copy(acc_sh.at[pl.ds(rbase, _ROWS_MAIN)],
                    out_hbm.at[c, pl.ds(rbase, _ROWS_MAIN)])

    @pl.when(s == _NS - 1)
    def _copy_tail():
        pltpu.sync_copy(
            acc_sh.at[pl.ds(rbase + _ROWS_MAIN, _ROWS_LAST - _ROWS_MAIN)],
            out_hbm.at[c, pl.ds(rbase + _ROWS_MAIN, _ROWS_LAST - _ROWS_MAIN)])


_agg_call_cache = []


def _agg_call(x, src, dst):
    if not _agg_call_cache:
        _agg_call_cache.append(functools.partial(
            pl.kernel,
            out_type=jax.ShapeDtypeStruct((_NC, _N, _D), jnp.float32),
            mesh=plsc.VectorSubcoreMesh(
                core_axis_name="c", subcore_axis_name="s",
                num_cores=_NC, num_subcores=_NS),
            scratch_types=[
                pltpu.VMEM_SHARED((_N, _D), jnp.float32),
                pltpu.VMEM((_CH,), jnp.int32),
                pltpu.VMEM((_CH,), jnp.int32),
                pltpu.VMEM((_CH, _D), jnp.float32),
                pltpu.VMEM((_ZROWS, _D), jnp.float32),
                pltpu.SemaphoreType.DMA,
            ],
        )(_agg_body))
    return _agg_call_cache[0](x, src, dst)


def _mlp_body(x_ref, agg_ref, w1_ref, b1_ref, g_ref, be_ref, w2_ref, b2_ref,
              o_ref):
    h = x_ref[...] + agg_ref[0] + agg_ref[1]
    z = jnp.dot(h, w1_ref[...], preferred_element_type=jnp.float32)
    z = z + b1_ref[...]
    mu = jnp.mean(z, axis=0, keepdims=True)
    d = z - mu
    var = jnp.mean(d * d, axis=0, keepdims=True)
    zn = d * lax.rsqrt(var + 1e-5) * g_ref[...] + be_ref[...]
    zr = jnp.maximum(zn, 0.0)
    o_ref[...] = jnp.dot(zr, w2_ref[...],
                         preferred_element_type=jnp.float32) + b2_ref[...]


def kernel(x, edge_index, W1, b1, gamma, beta, W2, b2):
    agg = _agg_call(x, edge_index[0], edge_index[1])
    hid = W1.shape[1]
    return pl.pallas_call(
        _mlp_body,
        out_shape=jax.ShapeDtypeStruct((x.shape[0], W2.shape[1]), jnp.float32),
    )(x, agg, W1, b1.reshape(1, hid), gamma.reshape(1, hid),
      beta.reshape(1, hid), W2, b2.reshape(1, W2.shape[1]))
